# baseline (speedup 1.0000x reference)
import functools

import jax
import jax.numpy as jnp
from jax import lax
from jax.experimental import pallas as pl
from jax.experimental.pallas import tpu as pltpu
from jax.experimental.pallas import tpu_sc as plsc

D_MODEL = 64
DPAD = 128
CHUNK = 128
NC, NS = 2, 16
NW = NC * NS
NBUF = 4


@functools.cache
def _make_kernel(n_rows: int):
    n_chunks = n_rows // CHUNK
    cpw = n_chunks // NW
    n_grp = cpw // NBUF
    assert n_chunks % NW == 0 and cpw % NBUF == 0 and n_grp > 2
    mesh = plsc.VectorSubcoreMesh(core_axis_name="c", subcore_axis_name="s")

    @functools.partial(
        pl.kernel,
        out_type=jax.ShapeDtypeStruct((n_rows // 2, DPAD), jnp.float32),
        mesh=mesh,
        scratch_types=[
            pltpu.VMEM((cpw, CHUNK), jnp.int32),
            pltpu.VMEM((NBUF, CHUNK, DPAD), jnp.float32),
            pltpu.VMEM((NBUF, CHUNK // 2, DPAD), jnp.float32),
        ]
        + [pltpu.SemaphoreType.DMA] * (2 * NBUF),
    )
    def gather_kernel(idx_hbm, table_hbm, out_hbm, idx_v, rows_v, pack_v,
                      *sems):
        gsem = sems[:NBUF]
        osem = sems[NBUF:]
        wid = lax.axis_index("s") * NC + lax.axis_index("c")
        base = wid * cpw
        pltpu.sync_copy(idx_hbm.at[pl.ds(base, cpw)], idx_v)

        def fire(j, slot):
            pltpu.async_copy(table_hbm.at[idx_v.at[j]], rows_v.at[slot],
                             gsem[slot])

        def wait_gather(slot):
            pltpu.make_async_copy(table_hbm.at[pl.ds(0, CHUNK)],
                                  rows_v.at[slot], gsem[slot]).wait()

        def put(j, slot):
            pltpu.async_copy(pack_v.at[slot],
                             out_hbm.at[pl.ds((base + j) * (CHUNK // 2),
                                              CHUNK // 2)], osem[slot])

        def wait_put(slot):
            pltpu.make_async_copy(pack_v.at[slot],
                                  out_hbm.at[pl.ds(0, CHUNK // 2)],
                                  osem[slot]).wait()

        def pack(slot):
            def pb(p, _):
                for k in range(4):
                    pack_v[slot, p, pl.ds(k * 16, 16)] = (
                        rows_v[slot, 2 * p, pl.ds(k * 16, 16)])
                    pack_v[slot, p, pl.ds(64 + k * 16, 16)] = (
                        rows_v[slot, 2 * p + 1, pl.ds(k * 16, 16)])
                return 0

            lax.fori_loop(0, CHUNK // 2, pb, 0)

        for slot in range(NBUF):
            fire(slot, slot)

        for slot in range(NBUF):
            wait_gather(slot)
            pack(slot)
            fire(NBUF + slot, slot)
            put(slot, slot)

        def body(g, _):
            for slot in range(NBUF):
                j = g * NBUF + slot
                wait_gather(slot)
                wait_put(slot)
                pack(slot)
                fire(j + NBUF, slot)
                put(j, slot)
            return 0

        lax.fori_loop(1, n_grp - 1, body, 0)

        for slot in range(NBUF):
            j = (n_grp - 1) * NBUF + slot
            wait_gather(slot)
            wait_put(slot)
            pack(slot)
            put(j, slot)

        for slot in range(NBUF):
            wait_put(slot)

    return gather_kernel


def kernel(x, table):
    b, s = x.shape
    n_rows = b * s
    idx = x.astype(jnp.int32).reshape(n_rows // CHUNK, CHUNK)
    tpad = jnp.pad(table, ((0, 0), (0, DPAD - D_MODEL)))
    out2 = _make_kernel(n_rows)(idx, tpad)
    return out2.reshape(b, s, D_MODEL)


# speedup vs baseline: 1.0097x; 1.0097x over previous
import functools

import jax
import jax.numpy as jnp
from jax import lax
from jax.experimental import pallas as pl
from jax.experimental.pallas import tpu as pltpu
from jax.experimental.pallas import tpu_sc as plsc

D_MODEL = 64
DPAD = 128
CHUNK = 128
NC, NS = 2, 16
NW = NC * NS
NBUF = 4


@functools.cache
def _make_kernel(n_rows: int):
    n_chunks = n_rows // CHUNK
    cpw = n_chunks // NW
    n_grp = cpw // NBUF
    assert n_chunks % NW == 0 and cpw % NBUF == 0 and n_grp > 2
    mesh = plsc.VectorSubcoreMesh(core_axis_name="c", subcore_axis_name="s")

    @functools.partial(
        pl.kernel,
        out_type=jax.ShapeDtypeStruct((n_rows // 2, DPAD), jnp.float32),
        mesh=mesh,
        scratch_types=[
            pltpu.VMEM((cpw, CHUNK), jnp.int32),
            pltpu.VMEM((NBUF, CHUNK, DPAD), jnp.float32),
            pltpu.VMEM((NBUF, CHUNK // 2, DPAD), jnp.float32),
        ]
        + [pltpu.SemaphoreType.DMA] * (2 * NBUF),
    )
    def gather_kernel(idx_hbm, table_hbm, out_hbm, idx_v, rows_v, pack_v,
                      *sems):
        gsem = sems[:NBUF]
        osem = sems[NBUF:]
        wid = lax.axis_index("s") * NC + lax.axis_index("c")
        base = wid * cpw
        pltpu.sync_copy(idx_hbm.at[pl.ds(base, cpw)], idx_v)

        def fire(j, slot):
            pltpu.async_copy(table_hbm.at[idx_v.at[j]], rows_v.at[slot],
                             gsem[slot])

        def wait_gather(slot):
            pltpu.make_async_copy(table_hbm.at[pl.ds(0, CHUNK)],
                                  rows_v.at[slot], gsem[slot]).wait()

        def put(j, slot):
            pltpu.async_copy(pack_v.at[slot],
                             out_hbm.at[pl.ds((base + j) * (CHUNK // 2),
                                              CHUNK // 2)], osem[slot])

        def wait_put(slot):
            pltpu.make_async_copy(pack_v.at[slot],
                                  out_hbm.at[pl.ds(0, CHUNK // 2)],
                                  osem[slot]).wait()

        def pack(slot):
            h = CHUNK // 2

            @plsc.parallel_loop(0, h, unroll=8)
            def _pb(p):
                for k in range(4):
                    pack_v[slot, p, pl.ds(k * 16, 16)] = (
                        rows_v[slot, p, pl.ds(k * 16, 16)])
                    pack_v[slot, p, pl.ds(64 + k * 16, 16)] = (
                        rows_v[slot, h + p, pl.ds(k * 16, 16)])

        for slot in range(NBUF):
            fire(slot, slot)

        for slot in range(NBUF):
            wait_gather(slot)
            pack(slot)
            fire(NBUF + slot, slot)
            put(slot, slot)

        def body(g, _):
            for slot in range(NBUF):
                j = g * NBUF + slot
                wait_gather(slot)
                wait_put(slot)
                pack(slot)
                fire(j + NBUF, slot)
                put(j, slot)
            return 0

        lax.fori_loop(1, n_grp - 1, body, 0)

        for slot in range(NBUF):
            j = (n_grp - 1) * NBUF + slot
            wait_gather(slot)
            wait_put(slot)
            pack(slot)
            put(j, slot)

        for slot in range(NBUF):
            wait_put(slot)

    return gather_kernel


def kernel(x, table):
    b, s = x.shape
    n_rows = b * s
    idx = (x.astype(jnp.int32)
           .reshape(n_rows // CHUNK, CHUNK // 2, 2)
           .transpose(0, 2, 1)
           .reshape(n_rows // CHUNK, CHUNK))
    tpad = jnp.pad(table, ((0, 0), (0, DPAD - D_MODEL)))
    out2 = _make_kernel(n_rows)(idx, tpad)
    return out2.reshape(b, s, D_MODEL)


# speedup vs baseline: 1.3792x; 1.3660x over previous
import functools

import jax
import jax.numpy as jnp
from jax import lax
from jax.experimental import pallas as pl
from jax.experimental.pallas import tpu as pltpu
from jax.experimental.pallas import tpu_sc as plsc

D_MODEL = 64
DPAD = 128
CHUNK = 128
NC, NS = 2, 16
NW = NC * NS
NBUF = 4


@functools.cache
def _make_kernel(n_rows: int):
    n_chunks = n_rows // CHUNK
    cpw = n_chunks // NW
    n_grp = cpw // NBUF
    assert n_chunks % NW == 0 and cpw % NBUF == 0 and n_grp > 2
    mesh = plsc.VectorSubcoreMesh(core_axis_name="c", subcore_axis_name="s")

    @functools.partial(
        pl.kernel,
        out_type=jax.ShapeDtypeStruct((n_rows, DPAD), jnp.float32),
        mesh=mesh,
        scratch_types=[
            pltpu.VMEM((cpw * CHUNK,), jnp.int32),
            pltpu.VMEM((NBUF, CHUNK, DPAD), jnp.float32),
        ]
        + [pltpu.SemaphoreType.DMA] * (2 * NBUF),
    )
    def gather_kernel(idx_hbm, table_hbm, out_hbm, idx_v, rows_v, *sems):
        gsem = sems[:NBUF]
        osem = sems[NBUF:]
        wid = lax.axis_index("s") * NC + lax.axis_index("c")
        base = wid * cpw
        pltpu.sync_copy(idx_hbm.at[wid], idx_v)

        def fire(j, slot):
            pltpu.async_copy(table_hbm.at[idx_v.at[pl.ds(j * CHUNK, CHUNK)]],
                             rows_v.at[slot], gsem[slot])

        def wait_gather(slot):
            pltpu.make_async_copy(table_hbm.at[pl.ds(0, CHUNK)],
                                  rows_v.at[slot], gsem[slot]).wait()

        def put(j, slot):
            pltpu.async_copy(rows_v.at[slot],
                             out_hbm.at[pl.ds((base + j) * CHUNK, CHUNK)],
                             osem[slot])

        def wait_put(slot):
            pltpu.make_async_copy(rows_v.at[slot],
                                  out_hbm.at[pl.ds(0, CHUNK)],
                                  osem[slot]).wait()

        for slot in range(NBUF):
            fire(slot, slot)

        def body(g, _):
            for slot in range(NBUF):
                j = g * NBUF + slot
                wait_gather(slot)
                put(j, slot)
                wait_put(slot)
                fire(j + NBUF, slot)
            return 0

        lax.fori_loop(0, n_grp - 1, body, 0)

        for slot in range(NBUF):
            j = (n_grp - 1) * NBUF + slot
            wait_gather(slot)
            put(j, slot)
            wait_put(slot)

    return gather_kernel


def kernel(x, table):
    b, s = x.shape
    n_rows = b * s
    idx = x.astype(jnp.int32).reshape(NW, n_rows // NW)
    tpad = jnp.pad(table, ((0, 0), (0, DPAD - D_MODEL)))
    out = _make_kernel(n_rows)(idx, tpad)
    return out[:, :D_MODEL].reshape(b, s, D_MODEL)


# speedup vs baseline: 1.3833x; 1.0030x over previous
import functools

import jax
import jax.numpy as jnp
from jax import lax
from jax.experimental import pallas as pl
from jax.experimental.pallas import tpu as pltpu
from jax.experimental.pallas import tpu_sc as plsc

D_MODEL = 64
DPAD = 128
CHUNK = 128
NC, NS = 2, 16
NW = NC * NS
NBUF = 5


@functools.cache
def _make_kernel(n_rows: int):
    n_chunks = n_rows // CHUNK
    cpw = n_chunks // NW
    n_grp = cpw // NBUF
    assert n_chunks % NW == 0 and cpw % NBUF == 0 and n_grp > 2
    mesh = plsc.VectorSubcoreMesh(core_axis_name="c", subcore_axis_name="s")

    @functools.partial(
        pl.kernel,
        out_type=jax.ShapeDtypeStruct((n_rows, DPAD), jnp.float32),
        mesh=mesh,
        scratch_types=[
            pltpu.VMEM((cpw * CHUNK,), jnp.int32),
            pltpu.VMEM((NBUF, CHUNK, DPAD), jnp.float32),
        ]
        + [pltpu.SemaphoreType.DMA] * (2 * NBUF),
    )
    def gather_kernel(idx_hbm, table_hbm, out_hbm, idx_v, rows_v, *sems):
        gsem = sems[:NBUF]
        osem = sems[NBUF:]
        wid = lax.axis_index("s") * NC + lax.axis_index("c")
        base = wid * cpw
        pltpu.sync_copy(idx_hbm.at[wid], idx_v)

        def fire(j, slot):
            pltpu.async_copy(table_hbm.at[idx_v.at[pl.ds(j * CHUNK, CHUNK)]],
                             rows_v.at[slot], gsem[slot])

        def wait_gather(slot):
            pltpu.make_async_copy(table_hbm.at[pl.ds(0, CHUNK)],
                                  rows_v.at[slot], gsem[slot]).wait()

        def put(j, slot):
            pltpu.async_copy(rows_v.at[slot],
                             out_hbm.at[pl.ds((base + j) * CHUNK, CHUNK)],
                             osem[slot])

        def wait_put(slot):
            pltpu.make_async_copy(rows_v.at[slot],
                                  out_hbm.at[pl.ds(0, CHUNK)],
                                  osem[slot]).wait()

        for slot in range(NBUF):
            fire(slot, slot)

        def body(g, _):
            for slot in range(NBUF):
                j = g * NBUF + slot
                wait_gather(slot)
                put(j, slot)
                wait_put(slot)
                fire(j + NBUF, slot)
            return 0

        lax.fori_loop(0, n_grp - 1, body, 0)

        for slot in range(NBUF):
            j = (n_grp - 1) * NBUF + slot
            wait_gather(slot)
            put(j, slot)
            wait_put(slot)

    return gather_kernel


def kernel(x, table):
    b, s = x.shape
    n_rows = b * s
    idx = x.astype(jnp.int32).reshape(NW, n_rows // NW)
    tpad = jnp.pad(table, ((0, 0), (0, DPAD - D_MODEL)))
    out = _make_kernel(n_rows)(idx, tpad)
    return out[:, :D_MODEL].reshape(b, s, D_MODEL)
